# baseline (speedup 1.0000x reference)
import functools

import jax
import jax.numpy as jnp
import numpy as np
from jax import lax
from jax.experimental import pallas as pl
from jax.experimental.pallas import tpu as pltpu
from jax.experimental.pallas import tpu_sc as plsc

D_MODEL = 768
VOCAB = 100000
MAX_POS = 2048
BATCH = 4
SEQ = 2048

NC, NS, LANES = 2, 16, 16
NW = NC * NS
SPW = SEQ // NW
NGRP = D_MODEL // (2 * LANES)

CHUNK = 32
NCHUNK = BATCH * SPW // CHUNK
NBUF = 4
AHEAD = 2

SCALE = float(np.float32(np.sqrt(np.float32(D_MODEL))))


def _pos_encoding_np():
    pos = np.arange(MAX_POS)[:, np.newaxis]
    i = np.arange(D_MODEL)[np.newaxis, :]
    angle_rates = 1 / np.power(10000, 2 * i // np.float32(D_MODEL))
    angle_rads = pos * angle_rates
    angle_rads[:, 0::2] = np.sin(angle_rads[:, 0::2])
    angle_rads[:, 1::2] = np.cos(angle_rads[:, 1::2])
    return angle_rads.astype(np.float32)


def _pos_bf16_interleaved_np():
    p = _pos_encoding_np().reshape(MAX_POS, NGRP, 2, LANES)
    p = np.swapaxes(p, 2, 3).reshape(MAX_POS, D_MODEL)
    p = p.reshape(-1).astype(jnp.bfloat16)
    return p.view(np.int32)


_MESH = plsc.VectorSubcoreMesh(core_axis_name="c", subcore_axis_name="s")


@functools.partial(
    pl.kernel,
    out_type=jax.ShapeDtypeStruct((BATCH, SEQ, D_MODEL), jnp.float32),
    mesh=_MESH,
    scratch_types=[
        pltpu.VMEM((BATCH, SPW), jnp.int32),
        pltpu.VMEM((NBUF, CHUNK, D_MODEL), jnp.float32),
        pltpu.VMEM((SPW * D_MODEL // 2,), jnp.int32),
        pltpu.SemaphoreType.DMA,
    ] + [pltpu.SemaphoreType.DMA] * (2 * NBUF),
)
def _emb_kernel(idx_hbm, table_hbm, pos_hbm, out_hbm,
                idx_v, rows_v, pos_i, psem, *sems):
    gsem, ssem = sems[:NBUF], sems[NBUF:]
    wid = lax.axis_index("s") * NC + lax.axis_index("c")
    base = wid * SPW

    def start_gather(i):
        b, off = divmod(i * CHUNK, SPW)
        return pltpu.async_copy(
            table_hbm.at[idx_v.at[b, pl.ds(off, CHUNK)]],
            rows_v.at[i % NBUF], gsem[i % NBUF])

    pltpu.sync_copy(idx_hbm.at[0, pl.ds(base, SPW)], idx_v.at[0])

    gathers = [None] * NCHUNK
    stores = [None] * NCHUNK
    chunks_per_batch = SPW // CHUNK
    for i in range(min(AHEAD, chunks_per_batch)):
        gathers[i] = start_gather(i)

    for b in range(1, BATCH):
        pltpu.sync_copy(idx_hbm.at[b, pl.ds(base, SPW)], idx_v.at[b])
    for i in range(chunks_per_batch, min(AHEAD, NCHUNK)):
        gathers[i] = start_gather(i)

    pos_copy = pltpu.async_copy(
        pos_hbm.at[pl.ds(pl.multiple_of(base * (D_MODEL // 2), 8), SPW * D_MODEL // 2)], pos_i, psem)
    pos_copy.wait()

    for i in range(NCHUNK):
        gathers[i].wait()
        b, off = divmod(i * CHUNK, SPW)
        buf = rows_v.at[i % NBUF]

        @plsc.parallel_loop(0, CHUNK)
        def row_body(r, buf=buf, off=off):
            for g in range(NGRP):
                pbase = pl.multiple_of((r + off) * (D_MODEL // 2), 8)
                pi = pos_i[pl.ds(pbase + g * 16, 16)]
                lo = lax.bitcast_convert_type(
                    lax.shift_left(pi, 16), jnp.float32)
                hi = lax.bitcast_convert_type(
                    lax.bitwise_and(pi, jnp.int32(-65536)), jnp.float32)
                pp = (lo, hi)
                for h in range(2):
                    sl = pl.ds(g * 32 + h * LANES, LANES)
                    buf[r, sl] = buf[r, sl] * SCALE + pp[h]

        stores[i] = pltpu.async_copy(
            buf, out_hbm.at[b, pl.ds(base + off, CHUNK), :], ssem[i % NBUF])

        nxt = i + AHEAD
        if nxt < NCHUNK:
            prev = nxt - NBUF
            if prev >= 0:
                stores[prev].wait()
                stores[prev] = None
            gathers[nxt] = start_gather(nxt)

    for s in stores:
        if s is not None:
            s.wait()


def kernel(inputs, table):
    pos = jnp.asarray(_pos_bf16_interleaved_np())
    return _emb_kernel(inputs, table, pos)


# speedup vs baseline: 1.0120x; 1.0120x over previous
import functools

import jax
import jax.numpy as jnp
import numpy as np
from jax import lax
from jax.experimental import pallas as pl
from jax.experimental.pallas import tpu as pltpu
from jax.experimental.pallas import tpu_sc as plsc

D_MODEL = 768
VOCAB = 100000
MAX_POS = 2048
BATCH = 4
SEQ = 2048

NC, NS, LANES = 2, 16, 16
NW = NC * NS
SPW = SEQ // NW
NGRP = D_MODEL // (2 * LANES)

CHUNK = 32
NCHUNK = BATCH * SPW // CHUNK
NBUF = 4
AHEAD = 3

SCALE = float(np.float32(np.sqrt(np.float32(D_MODEL))))


def _pos_encoding_np():
    pos = np.arange(MAX_POS)[:, np.newaxis]
    i = np.arange(D_MODEL)[np.newaxis, :]
    angle_rates = 1 / np.power(10000, 2 * i // np.float32(D_MODEL))
    angle_rads = pos * angle_rates
    angle_rads[:, 0::2] = np.sin(angle_rads[:, 0::2])
    angle_rads[:, 1::2] = np.cos(angle_rads[:, 1::2])
    return angle_rads.astype(np.float32)


def _pos_bf16_interleaved_np():
    p = _pos_encoding_np().reshape(MAX_POS, NGRP, 2, LANES)
    p = np.swapaxes(p, 2, 3).reshape(MAX_POS, D_MODEL)
    p = p.reshape(-1).astype(jnp.bfloat16)
    return p.view(np.int32)


_MESH = plsc.VectorSubcoreMesh(core_axis_name="c", subcore_axis_name="s")


@functools.partial(
    pl.kernel,
    out_type=jax.ShapeDtypeStruct((BATCH, SEQ, D_MODEL), jnp.float32),
    mesh=_MESH,
    scratch_types=[
        pltpu.VMEM((BATCH, SPW), jnp.int32),
        pltpu.VMEM((NBUF, CHUNK, D_MODEL), jnp.float32),
        pltpu.VMEM((SPW * D_MODEL // 2,), jnp.int32),
        pltpu.SemaphoreType.DMA,
    ] + [pltpu.SemaphoreType.DMA] * (2 * NBUF),
)
def _emb_kernel(idx_hbm, table_hbm, pos_hbm, out_hbm,
                idx_v, rows_v, pos_i, psem, *sems):
    gsem, ssem = sems[:NBUF], sems[NBUF:]
    wid = lax.axis_index("s") * NC + lax.axis_index("c")
    base = wid * SPW

    def start_gather(i):
        b, off = divmod(i * CHUNK, SPW)
        return pltpu.async_copy(
            table_hbm.at[idx_v.at[b, pl.ds(off, CHUNK)]],
            rows_v.at[i % NBUF], gsem[i % NBUF])

    pltpu.sync_copy(idx_hbm.at[0, pl.ds(base, SPW)], idx_v.at[0])

    gathers = [None] * NCHUNK
    stores = [None] * NCHUNK
    chunks_per_batch = SPW // CHUNK
    for i in range(min(AHEAD, chunks_per_batch)):
        gathers[i] = start_gather(i)

    for b in range(1, BATCH):
        pltpu.sync_copy(idx_hbm.at[b, pl.ds(base, SPW)], idx_v.at[b])
    for i in range(chunks_per_batch, min(AHEAD, NCHUNK)):
        gathers[i] = start_gather(i)

    pos_copy = pltpu.async_copy(
        pos_hbm.at[pl.ds(pl.multiple_of(base * (D_MODEL // 2), 8), SPW * D_MODEL // 2)], pos_i, psem)
    pos_copy.wait()

    for i in range(NCHUNK):
        gathers[i].wait()
        b, off = divmod(i * CHUNK, SPW)
        buf = rows_v.at[i % NBUF]

        @plsc.parallel_loop(0, CHUNK)
        def row_body(r, buf=buf, off=off):
            for g in range(NGRP):
                pbase = pl.multiple_of((r + off) * (D_MODEL // 2), 8)
                pi = pos_i[pl.ds(pbase + g * 16, 16)]
                lo = lax.bitcast_convert_type(
                    lax.shift_left(pi, 16), jnp.float32)
                hi = lax.bitcast_convert_type(
                    lax.bitwise_and(pi, jnp.int32(-65536)), jnp.float32)
                pp = (lo, hi)
                for h in range(2):
                    sl = pl.ds(g * 32 + h * LANES, LANES)
                    buf[r, sl] = buf[r, sl] * SCALE + pp[h]

        stores[i] = pltpu.async_copy(
            buf, out_hbm.at[b, pl.ds(base + off, CHUNK), :], ssem[i % NBUF])

        nxt = i + AHEAD
        if nxt < NCHUNK:
            prev = nxt - NBUF
            if prev >= 0:
                stores[prev].wait()
                stores[prev] = None
            gathers[nxt] = start_gather(nxt)

    for s in stores:
        if s is not None:
            s.wait()


def kernel(inputs, table):
    pos = jnp.asarray(_pos_bf16_interleaved_np())
    return _emb_kernel(inputs, table, pos)


# speedup vs baseline: 1.0397x; 1.0274x over previous
import functools

import jax
import jax.numpy as jnp
import numpy as np
from jax import lax
from jax.experimental import pallas as pl
from jax.experimental.pallas import tpu as pltpu
from jax.experimental.pallas import tpu_sc as plsc

D_MODEL = 768
VOCAB = 100000
MAX_POS = 2048
BATCH = 4
SEQ = 2048

NC, NS, LANES = 2, 16, 16
NW = NC * NS
SPW = SEQ // NW
NGRP = D_MODEL // (2 * LANES)

CHUNK = 32
NCHUNK = BATCH * SPW // CHUNK
NBUF = 4
AHEAD = 3

SCALE = float(np.float32(np.sqrt(np.float32(D_MODEL))))


def _pos_encoding_np():
    pos = np.arange(MAX_POS)[:, np.newaxis]
    i = np.arange(D_MODEL)[np.newaxis, :]
    angle_rates = 1 / np.power(10000, 2 * i // np.float32(D_MODEL))
    angle_rads = pos * angle_rates
    angle_rads[:, 0::2] = np.sin(angle_rads[:, 0::2])
    angle_rads[:, 1::2] = np.cos(angle_rads[:, 1::2])
    return angle_rads.astype(np.float32)


def _pos_bf16_interleaved_np():
    p = _pos_encoding_np().reshape(MAX_POS, NGRP, 2, LANES)
    p = np.swapaxes(p, 2, 3).reshape(MAX_POS, D_MODEL)
    p = p.reshape(-1).astype(jnp.bfloat16)
    return p.view(np.int32)


_MESH = plsc.VectorSubcoreMesh(core_axis_name="c", subcore_axis_name="s")


@functools.partial(
    pl.kernel,
    out_type=jax.ShapeDtypeStruct((BATCH, SEQ, D_MODEL), jnp.float32),
    mesh=_MESH,
    scratch_types=[
        pltpu.VMEM((BATCH, SPW), jnp.int32),
        pltpu.VMEM((NBUF, CHUNK, D_MODEL), jnp.float32),
        pltpu.VMEM((SPW * D_MODEL // 2,), jnp.int32),
        pltpu.SemaphoreType.DMA,
    ] + [pltpu.SemaphoreType.DMA] * (2 * NBUF),
)
def _emb_kernel(idx_hbm, table_hbm, pos_hbm, out_hbm,
                idx_v, rows_v, pos_i, psem, *sems):
    gsem, ssem = sems[:NBUF], sems[NBUF:]
    wid = lax.axis_index("s") * NC + lax.axis_index("c")
    base = wid * SPW

    def start_gather(i):
        b, off = divmod(i * CHUNK, SPW)
        return pltpu.async_copy(
            table_hbm.at[idx_v.at[b, pl.ds(off, CHUNK)]],
            rows_v.at[i % NBUF], gsem[i % NBUF])

    pltpu.sync_copy(idx_hbm.at[0, pl.ds(base, SPW)], idx_v.at[0])

    gathers = [None] * NCHUNK
    stores = [None] * NCHUNK
    chunks_per_batch = SPW // CHUNK
    for i in range(min(AHEAD, chunks_per_batch)):
        gathers[i] = start_gather(i)

    pos_copy = pltpu.async_copy(
        pos_hbm.at[pl.ds(pl.multiple_of(base * (D_MODEL // 2), 8),
                         SPW * D_MODEL // 2)], pos_i, psem)
    for b in range(1, BATCH):
        pltpu.sync_copy(idx_hbm.at[b, pl.ds(base, SPW)], idx_v.at[b])
    for i in range(chunks_per_batch, min(AHEAD, NCHUNK)):
        gathers[i] = start_gather(i)
    pos_copy.wait()

    for i in range(NCHUNK):
        gathers[i].wait()
        b, off = divmod(i * CHUNK, SPW)
        buf = rows_v.at[i % NBUF]

        @plsc.parallel_loop(0, CHUNK)
        def row_body(r, buf=buf, off=off):
            for g in range(NGRP):
                pbase = pl.multiple_of((r + off) * (D_MODEL // 2), 8)
                pi = pos_i[pl.ds(pbase + g * 16, 16)]
                lo = lax.bitcast_convert_type(
                    lax.shift_left(pi, 16), jnp.float32)
                hi = lax.bitcast_convert_type(
                    lax.bitwise_and(pi, jnp.int32(-65536)), jnp.float32)
                pp = (lo, hi)
                for h in range(2):
                    sl = pl.ds(g * 32 + h * LANES, LANES)
                    buf[r, sl] = buf[r, sl] * SCALE + pp[h]

        stores[i] = pltpu.async_copy(
            buf, out_hbm.at[b, pl.ds(base + off, CHUNK), :], ssem[i % NBUF])

        nxt = i + AHEAD
        if nxt < NCHUNK:
            prev = nxt - NBUF
            if prev >= 0:
                stores[prev].wait()
                stores[prev] = None
            gathers[nxt] = start_gather(nxt)

    for s in stores:
        if s is not None:
            s.wait()


def kernel(inputs, table):
    pos = jnp.asarray(_pos_bf16_interleaved_np())
    return _emb_kernel(inputs, table, pos)


# speedup vs baseline: 1.0496x; 1.0095x over previous
import functools

import jax
import jax.numpy as jnp
import numpy as np
from jax import lax
from jax.experimental import pallas as pl
from jax.experimental.pallas import tpu as pltpu
from jax.experimental.pallas import tpu_sc as plsc

D_MODEL = 768
VOCAB = 100000
MAX_POS = 2048
BATCH = 4
SEQ = 2048

NC, NS, LANES = 2, 16, 16
NW = NC * NS
SPW = SEQ // NW
NGRP = D_MODEL // (2 * LANES)

CHUNK = 32
NCHUNK = BATCH * SPW // CHUNK
NBUF = 4
AHEAD = 3

SCALE = float(np.float32(np.sqrt(np.float32(D_MODEL))))


def _pos_encoding_np():
    pos = np.arange(MAX_POS)[:, np.newaxis]
    i = np.arange(D_MODEL)[np.newaxis, :]
    angle_rates = 1 / np.power(10000, 2 * i // np.float32(D_MODEL))
    angle_rads = pos * angle_rates
    angle_rads[:, 0::2] = np.sin(angle_rads[:, 0::2])
    angle_rads[:, 1::2] = np.cos(angle_rads[:, 1::2])
    return angle_rads.astype(np.float32)


def _pos_bf16_interleaved_np():
    p = _pos_encoding_np().reshape(MAX_POS, NGRP, 2, LANES)
    p = np.swapaxes(p, 2, 3).reshape(MAX_POS, D_MODEL)
    p = p.reshape(-1).astype(jnp.bfloat16)
    return p.view(np.int32)


_MESH = plsc.VectorSubcoreMesh(core_axis_name="c", subcore_axis_name="s")


@functools.partial(
    pl.kernel,
    out_type=jax.ShapeDtypeStruct((BATCH, SEQ, D_MODEL), jnp.float32),
    mesh=_MESH,
    scratch_types=[
        pltpu.VMEM((BATCH, SPW), jnp.int32),
        pltpu.VMEM((NBUF, CHUNK, D_MODEL), jnp.float32),
        pltpu.VMEM((SPW * D_MODEL // 2,), jnp.int32),
        pltpu.SemaphoreType.DMA,
    ] + [pltpu.SemaphoreType.DMA] * (2 * NBUF),
)
def _emb_kernel(idx_hbm, table_hbm, pos_hbm, out_hbm,
                idx_v, rows_v, pos_i, psem, *sems):
    gsem, ssem = sems[:NBUF], sems[NBUF:]
    wid = lax.axis_index("s") * NC + lax.axis_index("c")
    base = wid * SPW

    def start_gather(i):
        b, off = divmod(i * CHUNK, SPW)
        return pltpu.async_copy(
            table_hbm.at[idx_v.at[b, pl.ds(off, CHUNK)]],
            rows_v.at[i % NBUF], gsem[i % NBUF])

    pos_copy = pltpu.async_copy(
        pos_hbm.at[pl.ds(pl.multiple_of(base * (D_MODEL // 2), 8),
                         SPW * D_MODEL // 2)], pos_i, psem)
    pltpu.sync_copy(idx_hbm.at[0, pl.ds(base, SPW)], idx_v.at[0])

    gathers = [None] * NCHUNK
    stores = [None] * NCHUNK
    chunks_per_batch = SPW // CHUNK
    for i in range(min(AHEAD, chunks_per_batch)):
        gathers[i] = start_gather(i)

    for b in range(1, BATCH):
        pltpu.sync_copy(idx_hbm.at[b, pl.ds(base, SPW)], idx_v.at[b])
    for i in range(chunks_per_batch, min(AHEAD, NCHUNK)):
        gathers[i] = start_gather(i)
    pos_copy.wait()

    for i in range(NCHUNK):
        gathers[i].wait()
        b, off = divmod(i * CHUNK, SPW)
        buf = rows_v.at[i % NBUF]

        @plsc.parallel_loop(0, CHUNK)
        def row_body(r, buf=buf, off=off):
            for g in range(NGRP):
                pbase = pl.multiple_of((r + off) * (D_MODEL // 2), 8)
                pi = pos_i[pl.ds(pbase + g * 16, 16)]
                lo = lax.bitcast_convert_type(
                    lax.shift_left(pi, 16), jnp.float32)
                hi = lax.bitcast_convert_type(
                    lax.bitwise_and(pi, jnp.int32(-65536)), jnp.float32)
                pp = (lo, hi)
                for h in range(2):
                    sl = pl.ds(g * 32 + h * LANES, LANES)
                    buf[r, sl] = buf[r, sl] * SCALE + pp[h]

        stores[i] = pltpu.async_copy(
            buf, out_hbm.at[b, pl.ds(base + off, CHUNK), :], ssem[i % NBUF])

        nxt = i + AHEAD
        if nxt < NCHUNK:
            prev = nxt - NBUF
            if prev >= 0:
                stores[prev].wait()
                stores[prev] = None
            gathers[nxt] = start_gather(nxt)

    for s in stores:
        if s is not None:
            s.wait()


def kernel(inputs, table):
    pos = jnp.asarray(_pos_bf16_interleaved_np())
    return _emb_kernel(inputs, table, pos)


# speedup vs baseline: 1.0576x; 1.0076x over previous
import functools

import jax
import jax.numpy as jnp
import numpy as np
from jax import lax
from jax.experimental import pallas as pl
from jax.experimental.pallas import tpu as pltpu
from jax.experimental.pallas import tpu_sc as plsc

D_MODEL = 768
VOCAB = 100000
MAX_POS = 2048
BATCH = 4
SEQ = 2048

NC, NS, LANES = 2, 16, 16
NW = NC * NS
SPW = SEQ // NW
NGRP = D_MODEL // (2 * LANES)

CHUNK = 32
NCHUNK = BATCH * SPW // CHUNK
NBUF = 4
AHEAD = 3

SCALE = float(np.float32(np.sqrt(np.float32(D_MODEL))))


def _pos_encoding_np():
    pos = np.arange(MAX_POS)[:, np.newaxis]
    i = np.arange(D_MODEL)[np.newaxis, :]
    angle_rates = 1 / np.power(10000, 2 * i // np.float32(D_MODEL))
    angle_rads = pos * angle_rates
    angle_rads[:, 0::2] = np.sin(angle_rads[:, 0::2])
    angle_rads[:, 1::2] = np.cos(angle_rads[:, 1::2])
    return angle_rads.astype(np.float32)


def _pos_bf16_interleaved_np():
    p = _pos_encoding_np().reshape(MAX_POS, NGRP, 2, LANES)
    p = np.swapaxes(p, 2, 3).reshape(MAX_POS, D_MODEL)
    p = p.reshape(-1).astype(jnp.bfloat16)
    return p.view(np.int32)


_MESH = plsc.VectorSubcoreMesh(core_axis_name="c", subcore_axis_name="s")


@functools.partial(
    pl.kernel,
    out_type=jax.ShapeDtypeStruct((BATCH, SEQ, D_MODEL), jnp.float32),
    mesh=_MESH,
    scratch_types=[
        pltpu.VMEM((BATCH, SPW), jnp.int32),
        pltpu.VMEM((NBUF, CHUNK, D_MODEL), jnp.float32),
        pltpu.VMEM((SPW * D_MODEL // 2,), jnp.int32),
        pltpu.SemaphoreType.DMA,
    ] + [pltpu.SemaphoreType.DMA] * (2 * NBUF),
)
def _emb_kernel(idx_hbm, table_hbm, pos_hbm, out_hbm,
                idx_v, rows_v, pos_i, psem, *sems):
    gsem, ssem = sems[:NBUF], sems[NBUF:]
    wid = lax.axis_index("s") * NC + lax.axis_index("c")
    base = wid * SPW

    def start_gather(i):
        b, off = divmod(i * CHUNK, SPW)
        return pltpu.async_copy(
            table_hbm.at[idx_v.at[b, pl.ds(off, CHUNK)]],
            rows_v.at[i % NBUF], gsem[i % NBUF])

    pos_copy = pltpu.async_copy(
        pos_hbm.at[pl.ds(pl.multiple_of(base * (D_MODEL // 2), 8),
                         SPW * D_MODEL // 2)], pos_i, psem)
    pltpu.sync_copy(idx_hbm.at[0, pl.ds(base, SPW)], idx_v.at[0])

    gathers = [None] * NCHUNK
    stores = [None] * NCHUNK
    chunks_per_batch = SPW // CHUNK
    for i in range(min(NBUF, chunks_per_batch)):
        gathers[i] = start_gather(i)

    for b in range(1, BATCH):
        pltpu.sync_copy(idx_hbm.at[b, pl.ds(base, SPW)], idx_v.at[b])
    for i in range(chunks_per_batch, min(NBUF, NCHUNK)):
        gathers[i] = start_gather(i)
    pos_copy.wait()

    for i in range(NCHUNK):
        gathers[i].wait()
        b, off = divmod(i * CHUNK, SPW)
        buf = rows_v.at[i % NBUF]

        @plsc.parallel_loop(0, CHUNK)
        def row_body(r, buf=buf, off=off):
            for g in range(NGRP):
                pbase = pl.multiple_of((r + off) * (D_MODEL // 2), 8)
                pi = pos_i[pl.ds(pbase + g * 16, 16)]
                lo = lax.bitcast_convert_type(
                    lax.shift_left(pi, 16), jnp.float32)
                hi = lax.bitcast_convert_type(
                    lax.bitwise_and(pi, jnp.int32(-65536)), jnp.float32)
                pp = (lo, hi)
                for h in range(2):
                    sl = pl.ds(g * 32 + h * LANES, LANES)
                    buf[r, sl] = buf[r, sl] * SCALE + pp[h]

        stores[i] = pltpu.async_copy(
            buf, out_hbm.at[b, pl.ds(base + off, CHUNK), :], ssem[i % NBUF])

        nxt = i + AHEAD
        if nxt < NCHUNK and gathers[nxt] is None:
            prev = nxt - NBUF
            if prev >= 0:
                stores[prev].wait()
                stores[prev] = None
            gathers[nxt] = start_gather(nxt)

    for s in stores:
        if s is not None:
            s.wait()


def kernel(inputs, table):
    pos = jnp.asarray(_pos_bf16_interleaved_np())
    return _emb_kernel(inputs, table, pos)
